# baseline (speedup 1.0000x reference)
import functools

import jax
import jax.numpy as jnp
from jax import lax
from jax.experimental import pallas as pl
from jax.experimental.pallas import tpu as pltpu
from jax.experimental.pallas import tpu_sc as plsc

VOCAB = 1000
EMB_DIM = 128
HID = 8
HID_P = 16
ROW = 128
VPAD = 1024
NC = 2
NS = 16
NW = NC * NS
CHUNK = 128
TILE = 2048
VMAIN = 896
VTAIL = VOCAB - VMAIN


def _table_body(embp_ref, wa_ref, wb_ref, t_ref):
    dn = (((1,), (1,)), ((), ()))
    t_ref[0:VPAD, :] = lax.dot_general(
        embp_ref[...], wa_ref[...], dn, preferred_element_type=jnp.float32)
    t_ref[VPAD:2 * VPAD, :] = lax.dot_general(
        embp_ref[...], wb_ref[...], dn, preferred_element_type=jnp.float32)


def _build_table(embp, wa, wb):
    return pl.pallas_call(
        _table_body,
        out_shape=jax.ShapeDtypeStruct((2 * VPAD, ROW), jnp.float32),
    )(embp, wa, wb)


def _sc_gather(table, idx0, idx1, nt):
    bpw = nt // NW
    nch = bpw // CHUNK
    mesh = plsc.VectorSubcoreMesh(core_axis_name="c", subcore_axis_name="s")

    @functools.partial(
        pl.kernel, mesh=mesh,
        out_type=jax.ShapeDtypeStruct((nt, HID_P), jnp.float32),
        scratch_types=[
            pltpu.VMEM((nch, CHUNK), jnp.int32),
            pltpu.VMEM((nch, CHUNK), jnp.int32),
            pltpu.VMEM((CHUNK, ROW), jnp.float32),
            pltpu.VMEM((CHUNK, ROW), jnp.float32),
            pltpu.VMEM((bpw, HID_P), jnp.float32),
            pltpu.SemaphoreType.DMA,
        ],
    )
    def gather_k(table_hbm, idx0_hbm, idx1_hbm, out_hbm,
                 i0_v, i1_v, g0_v, g1_v, h_v, sem):
        wid = lax.axis_index("s") * NC + lax.axis_index("c")
        pltpu.sync_copy(idx0_hbm.at[pl.ds(wid * nch, nch)], i0_v)
        pltpu.sync_copy(idx1_hbm.at[pl.ds(wid * nch, nch)], i1_v)
        for j in range(nch):
            c0 = pltpu.async_copy(table_hbm.at[i0_v.at[j]], g0_v, sem)
            c1 = pltpu.async_copy(table_hbm.at[i1_v.at[j]], g1_v, sem)
            c0.wait()
            c1.wait()

            def body(i, carry, j=j):
                h_v[j * CHUNK + i, :] = g0_v[i, 0:HID_P] + g1_v[i, 0:HID_P]
                return carry

            lax.fori_loop(0, CHUNK, body, 0)
        pltpu.sync_copy(h_v, out_hbm.at[pl.ds(wid * bpw, bpw)])

    return gather_k(table, idx0, idx1)


def _mlp_body(h_ref, w2_ref, b1_ref, b2_ref, out_ref):
    ht = jnp.tanh(h_ref[...] + b1_ref[...])
    dn = (((1,), (1,)), ((), ()))
    acc = lax.dot_general(ht, w2_ref[...], dn, preferred_element_type=jnp.float32)
    out_ref[...] = (acc + b2_ref[...]).astype(jnp.bfloat16)


def kernel(x, emb, fc1_w, fc1_b, fc2_w, fc2_b):
    x = x.astype(jnp.int32)
    batch = x.shape[0]

    embp = jnp.pad(emb, ((0, VPAD - VOCAB), (0, 0)))
    w1p = jnp.pad(fc1_w, ((0, ROW - HID), (0, 0)))
    table = _build_table(embp, w1p[:, :EMB_DIM], w1p[:, EMB_DIM:])

    idx0 = x[:, 0].reshape(batch // CHUNK, CHUNK)
    idx1 = (x[:, 1] + VPAD).reshape(batch // CHUNK, CHUNK)
    h = _sc_gather(table, idx0, idx1, batch)

    w2p = jnp.pad(fc2_w, ((0, 0), (0, HID_P - HID)))
    b1p = jnp.pad(fc1_b, (0, HID_P - HID)).reshape(1, HID_P)
    b2 = fc2_b.reshape(1, VOCAB)

    out1024 = pl.pallas_call(
        _mlp_body,
        grid=(batch // TILE,),
        in_specs=[
            pl.BlockSpec((TILE, HID_P), lambda i: (i, 0)),
            pl.BlockSpec((VPAD, HID_P), lambda i: (0, 0)),
            pl.BlockSpec((1, HID_P), lambda i: (0, 0)),
            pl.BlockSpec((1, VPAD), lambda i: (0, 0)),
        ],
        out_specs=pl.BlockSpec((TILE, VPAD), lambda i: (i, 0)),
        out_shape=jax.ShapeDtypeStruct((batch, VPAD), jnp.bfloat16),
    )(h, jnp.pad(w2p, ((0, VPAD - VOCAB), (0, 0))),
      b1p, jnp.pad(b2, ((0, 0), (0, VPAD - VOCAB))))
    return lax.slice(out1024, (0, 0), (batch, VOCAB)).astype(jnp.float32)


# speedup vs baseline: 1.2098x; 1.2098x over previous
import functools

import jax
import jax.numpy as jnp
from jax import lax
from jax.experimental import pallas as pl
from jax.experimental.pallas import tpu as pltpu
from jax.experimental.pallas import tpu_sc as plsc

VOCAB = 1000
EMB_DIM = 128
HID = 8
HID_P = 16
ROW = 128
VPAD = 1024
NC = 2
NS = 16
NW = NC * NS
CHUNK = 128
TILE = 2048
VMAIN = 896
VTAIL = VOCAB - VMAIN


def _table_body(embp_ref, wa_ref, wb_ref, t_ref):
    dn = (((1,), (1,)), ((), ()))
    t_ref[0:VPAD, :] = lax.dot_general(
        embp_ref[...], wa_ref[...], dn,
        preferred_element_type=jnp.float32)[:, 0:HID_P]
    t_ref[VPAD:2 * VPAD, :] = lax.dot_general(
        embp_ref[...], wb_ref[...], dn,
        preferred_element_type=jnp.float32)[:, 0:HID_P]


def _build_table(embp, wa, wb):
    return pl.pallas_call(
        _table_body,
        out_shape=jax.ShapeDtypeStruct((2 * VPAD, HID_P), jnp.float32),
    )(embp, wa, wb)


def _sc_gather(table, idx0, idx1, nt):
    bpw = nt // NW
    nch = bpw // CHUNK
    mesh = plsc.VectorSubcoreMesh(core_axis_name="c", subcore_axis_name="s")

    @functools.partial(
        pl.kernel, mesh=mesh,
        compiler_params=pltpu.CompilerParams(use_tc_tiling_on_sc=False),
        out_type=jax.ShapeDtypeStruct((nt, HID_P), jnp.float32),
        scratch_types=[
            pltpu.VMEM((nch, CHUNK), jnp.int32),
            pltpu.VMEM((nch, CHUNK), jnp.int32),
            pltpu.VMEM((CHUNK, HID_P), jnp.float32),
            pltpu.VMEM((CHUNK, HID_P), jnp.float32),
            pltpu.VMEM((bpw, HID_P), jnp.float32),
            pltpu.SemaphoreType.DMA,
        ],
    )
    def gather_k(table_hbm, idx0_hbm, idx1_hbm, out_hbm,
                 i0_v, i1_v, g0_v, g1_v, h_v, sem):
        wid = lax.axis_index("s") * NC + lax.axis_index("c")
        pltpu.sync_copy(idx0_hbm.at[pl.ds(wid * nch, nch)], i0_v)
        pltpu.sync_copy(idx1_hbm.at[pl.ds(wid * nch, nch)], i1_v)
        for j in range(nch):
            c0 = pltpu.async_copy(table_hbm.at[i0_v.at[j]], g0_v, sem)
            c1 = pltpu.async_copy(table_hbm.at[i1_v.at[j]], g1_v, sem)
            c0.wait()
            c1.wait()

            def body(i, carry, j=j):
                h_v[j * CHUNK + i, :] = g0_v[i, :] + g1_v[i, :]
                return carry

            lax.fori_loop(0, CHUNK, body, 0)
        pltpu.sync_copy(h_v, out_hbm.at[pl.ds(wid * bpw, bpw)])

    return gather_k(table, idx0, idx1)


def _mlp_body(h_ref, w2_ref, b1_ref, b2_ref, out_ref):
    ht = jnp.tanh(h_ref[...] + b1_ref[...])
    dn = (((1,), (1,)), ((), ()))
    acc = lax.dot_general(ht, w2_ref[...], dn, preferred_element_type=jnp.float32)
    out_ref[...] = acc + b2_ref[...]


def kernel(x, emb, fc1_w, fc1_b, fc2_w, fc2_b):
    x = x.astype(jnp.int32)
    batch = x.shape[0]

    embp = jnp.pad(emb, ((0, VPAD - VOCAB), (0, 0)))
    w1p = jnp.pad(fc1_w, ((0, ROW - HID), (0, 0)))
    table = _build_table(embp, w1p[:, :EMB_DIM], w1p[:, EMB_DIM:])

    idx0 = x[:, 0].reshape(batch // CHUNK, CHUNK)
    idx1 = (x[:, 1] + VPAD).reshape(batch // CHUNK, CHUNK)
    h = _sc_gather(table, idx0, idx1, batch)

    w2p = jnp.pad(fc2_w, ((0, 0), (0, HID_P - HID)))
    b1p = jnp.pad(fc1_b, (0, HID_P - HID)).reshape(1, HID_P)
    b2 = fc2_b.reshape(1, VOCAB)

    out1024 = pl.pallas_call(
        _mlp_body,
        grid=(batch // TILE,),
        in_specs=[
            pl.BlockSpec((TILE, HID_P), lambda i: (i, 0)),
            pl.BlockSpec((VPAD, HID_P), lambda i: (0, 0)),
            pl.BlockSpec((1, HID_P), lambda i: (0, 0)),
            pl.BlockSpec((1, VPAD), lambda i: (0, 0)),
        ],
        out_specs=pl.BlockSpec((TILE, VPAD), lambda i: (i, 0)),
        out_shape=jax.ShapeDtypeStruct((batch, VPAD), jnp.float32),
    )(h, jnp.pad(w2p, ((0, VPAD - VOCAB), (0, 0))),
      b1p, jnp.pad(b2, ((0, 0), (0, VPAD - VOCAB))))
    return lax.slice(out1024, (0, 0), (batch, VOCAB))


# speedup vs baseline: 1.2409x; 1.0257x over previous
import functools

import jax
import jax.numpy as jnp
from jax import lax
from jax.experimental import pallas as pl
from jax.experimental.pallas import tpu as pltpu
from jax.experimental.pallas import tpu_sc as plsc

VOCAB = 1000
EMB_DIM = 128
HID = 8
HID_P = 16
ROW = 128
VPAD = 1024
NC = 2
NS = 16
NW = NC * NS
CHUNK = 128
TILE = 2048
VMAIN = 896
VTAIL = VOCAB - VMAIN


def _table_body(embp_ref, wa_ref, wb_ref, t_ref):
    dn = (((1,), (1,)), ((), ()))
    t_ref[0:VPAD, :] = lax.dot_general(
        embp_ref[...], wa_ref[...], dn,
        preferred_element_type=jnp.float32)[:, 0:HID_P]
    t_ref[VPAD:2 * VPAD, :] = lax.dot_general(
        embp_ref[...], wb_ref[...], dn,
        preferred_element_type=jnp.float32)[:, 0:HID_P]


def _build_table(embp, wa, wb):
    return pl.pallas_call(
        _table_body,
        out_shape=jax.ShapeDtypeStruct((2 * VPAD, HID_P), jnp.float32),
    )(embp, wa, wb)


def _sc_gather(table, idx0, idx1, nt):
    bpw = nt // NW
    nch = bpw // CHUNK
    mesh = plsc.VectorSubcoreMesh(core_axis_name="c", subcore_axis_name="s")

    @functools.partial(
        pl.kernel, mesh=mesh,
        compiler_params=pltpu.CompilerParams(use_tc_tiling_on_sc=False),
        out_type=jax.ShapeDtypeStruct((nt, HID_P), jnp.float32),
        scratch_types=[
            pltpu.VMEM((nch, CHUNK), jnp.int32),
            pltpu.VMEM((nch, CHUNK), jnp.int32),
            pltpu.VMEM((bpw, HID_P), jnp.float32),
            pltpu.VMEM((bpw, HID_P), jnp.float32),
            pltpu.SemaphoreType.DMA,
        ],
    )
    def gather_k(table_hbm, idx0_hbm, idx1_hbm, out_hbm,
                 i0_v, i1_v, g0_v, g1_v, sem):
        wid = lax.axis_index("s") * NC + lax.axis_index("c")
        pltpu.sync_copy(idx0_hbm.at[pl.ds(wid * nch, nch)], i0_v)
        pltpu.sync_copy(idx1_hbm.at[pl.ds(wid * nch, nch)], i1_v)
        copies = []
        for j in range(nch):
            copies.append(pltpu.async_copy(
                table_hbm.at[i0_v.at[j]], g0_v.at[pl.ds(j * CHUNK, CHUNK)], sem))
            copies.append(pltpu.async_copy(
                table_hbm.at[i1_v.at[j]], g1_v.at[pl.ds(j * CHUNK, CHUNK)], sem))
        for c in copies:
            c.wait()

        def body(i, carry):
            for k in range(4):
                r = i * 4 + k
                g0_v[r, :] = g0_v[r, :] + g1_v[r, :]
            return carry

        lax.fori_loop(0, bpw // 4, body, 0)
        pltpu.sync_copy(g0_v, out_hbm.at[pl.ds(wid * bpw, bpw)])

    return gather_k(table, idx0, idx1)


def _mlp_body(h_ref, w2_ref, b1_ref, b2_ref, out_ref):
    ht = jnp.tanh(h_ref[...] + b1_ref[...])
    dn = (((1,), (1,)), ((), ()))
    acc = lax.dot_general(ht, w2_ref[...], dn, preferred_element_type=jnp.float32)
    out_ref[...] = acc + b2_ref[...]


def kernel(x, emb, fc1_w, fc1_b, fc2_w, fc2_b):
    x = x.astype(jnp.int32)
    batch = x.shape[0]

    embp = jnp.pad(emb, ((0, VPAD - VOCAB), (0, 0)))
    w1p = jnp.pad(fc1_w, ((0, ROW - HID), (0, 0)))
    table = _build_table(embp, w1p[:, :EMB_DIM], w1p[:, EMB_DIM:])

    idx0 = x[:, 0].reshape(batch // CHUNK, CHUNK)
    idx1 = (x[:, 1] + VPAD).reshape(batch // CHUNK, CHUNK)
    h = _sc_gather(table, idx0, idx1, batch)

    w2p = jnp.pad(fc2_w, ((0, 0), (0, HID_P - HID)))
    b1p = jnp.pad(fc1_b, (0, HID_P - HID)).reshape(1, HID_P)
    b2 = fc2_b.reshape(1, VOCAB)

    out1024 = pl.pallas_call(
        _mlp_body,
        grid=(batch // TILE,),
        in_specs=[
            pl.BlockSpec((TILE, HID_P), lambda i: (i, 0)),
            pl.BlockSpec((VPAD, HID_P), lambda i: (0, 0)),
            pl.BlockSpec((1, HID_P), lambda i: (0, 0)),
            pl.BlockSpec((1, VPAD), lambda i: (0, 0)),
        ],
        out_specs=pl.BlockSpec((TILE, VPAD), lambda i: (i, 0)),
        out_shape=jax.ShapeDtypeStruct((batch, VPAD), jnp.float32),
    )(h, jnp.pad(w2p, ((0, VPAD - VOCAB), (0, 0))),
      b1p, jnp.pad(b2, ((0, 0), (0, VPAD - VOCAB))))
    return lax.slice(out1024, (0, 0), (batch, VOCAB))
